# baseline (speedup 1.0000x reference)
import functools

import jax
import jax.numpy as jnp
from jax import lax
from jax.experimental import pallas as pl
from jax.experimental.pallas import tpu as pltpu
from jax.experimental.pallas import tpu_sc as plsc

_N_CAT = 26
_N_CONT = 13
_DIM = 32
_BATCH = 16384
_NC = 2
_NS = 16
_NW = _NC * _NS
_BPW = _BATCH // _NW
_GROUP = 6


def _group_kernel(nt, idx_hbm, *rest):
    tabs = rest[:nt]
    out = rest[nt]
    idx_v = rest[nt + 1]
    rows = rest[nt + 2:nt + 4]
    gsem = rest[nt + 4:nt + 6]
    wsem = rest[nt + 6:nt + 8]
    wid = lax.axis_index("s") * _NC + lax.axis_index("c")
    base = wid * _BPW

    pltpu.sync_copy(idx_hbm.at[:, pl.ds(base, _BPW)], idx_v)

    gd = [None, None]
    wd = [None, None]
    gd[0] = pltpu.async_copy(tabs[0].at[idx_v.at[0]], rows[0], gsem[0])
    for k in range(nt):
        b = k % 2
        if k + 1 < nt:
            if wd[1 - b] is not None:
                wd[1 - b].wait()
                wd[1 - b] = None
            gd[1 - b] = pltpu.async_copy(
                tabs[k + 1].at[idx_v.at[k + 1]], rows[1 - b], gsem[1 - b])
        gd[b].wait()
        wd[b] = pltpu.async_copy(
            rows[b], out.at[pl.ds(base, _BPW), pl.ds(k * _DIM, _DIM)],
            wsem[b])
    for b in range(min(nt, 2)):
        if wd[b] is not None:
            wd[b].wait()


@jax.jit
def _run(idx, *tabs):
    mesh = plsc.VectorSubcoreMesh(core_axis_name="c", subcore_axis_name="s")

    def make_group(nt):
        return functools.partial(
            pl.kernel,
            out_type=jax.ShapeDtypeStruct((_BATCH, nt * _DIM), jnp.float32),
            mesh=mesh,
            scratch_types=[
                pltpu.VMEM((nt, _BPW), jnp.int32),
                pltpu.VMEM((_BPW, _DIM), jnp.float32),
                pltpu.VMEM((_BPW, _DIM), jnp.float32),
                pltpu.SemaphoreType.DMA,
                pltpu.SemaphoreType.DMA,
                pltpu.SemaphoreType.DMA,
                pltpu.SemaphoreType.DMA,
            ],
            compiler_params=pltpu.CompilerParams(use_tc_tiling_on_sc=False),
        )(functools.partial(_group_kernel, nt))

    outs = []
    for g in range((_N_CAT + _GROUP - 1) // _GROUP):
        lo = g * _GROUP
        nt = min(_GROUP, _N_CAT - lo)
        outs.append(make_group(nt)(idx[lo:lo + nt], *tabs[lo:lo + nt]))
    return jnp.concatenate(outs, axis=1)


def kernel(x, table_0, table_1, table_2, table_3, table_4, table_5, table_6,
           table_7, table_8, table_9, table_10, table_11, table_12, table_13,
           table_14, table_15, table_16, table_17, table_18, table_19,
           table_20, table_21, table_22, table_23, table_24, table_25):
    tabs = (table_0, table_1, table_2, table_3, table_4, table_5, table_6,
            table_7, table_8, table_9, table_10, table_11, table_12, table_13,
            table_14, table_15, table_16, table_17, table_18, table_19,
            table_20, table_21, table_22, table_23, table_24, table_25)
    idx = x[:, :_N_CAT].astype(jnp.int32).T
    emb = _run(idx, *tabs)
    cont = x[:, _N_CAT:_N_CAT + _N_CONT]
    return emb, cont


# speedup vs baseline: 1.0819x; 1.0819x over previous
import functools

import jax
import jax.numpy as jnp
from jax import lax
from jax.experimental import pallas as pl
from jax.experimental.pallas import tpu as pltpu
from jax.experimental.pallas import tpu_sc as plsc

_N_CAT = 26
_N_CONT = 13
_DIM = 32
_BATCH = 16384
_NC = 2
_NS = 16
_NW = _NC * _NS
_BPW = _BATCH // _NW
_GROUP = 4


def _group_kernel(nt, idx_hbm, *rest):
    tabs = rest[:nt]
    out = rest[nt]
    idx_v = rest[nt + 1]
    rows = rest[nt + 2:nt + 4]
    gsem = rest[nt + 4:nt + 6]
    wsem = rest[nt + 6:nt + 8]
    wid = lax.axis_index("s") * _NC + lax.axis_index("c")
    base = wid * _BPW

    pltpu.sync_copy(idx_hbm.at[:, pl.ds(base, _BPW)], idx_v)

    gd = [None, None]
    wd = [None, None]
    gd[0] = pltpu.async_copy(tabs[0].at[idx_v.at[0]], rows[0], gsem[0])
    for k in range(nt):
        b = k % 2
        if k + 1 < nt:
            if wd[1 - b] is not None:
                wd[1 - b].wait()
                wd[1 - b] = None
            gd[1 - b] = pltpu.async_copy(
                tabs[k + 1].at[idx_v.at[k + 1]], rows[1 - b], gsem[1 - b])
        gd[b].wait()
        wd[b] = pltpu.async_copy(
            rows[b], out.at[pl.ds(base, _BPW), pl.ds(k * _DIM, _DIM)],
            wsem[b])
    for b in range(min(nt, 2)):
        if wd[b] is not None:
            wd[b].wait()


@jax.jit
def _run(idx, *tabs):
    mesh = plsc.VectorSubcoreMesh(core_axis_name="c", subcore_axis_name="s")

    def make_group(nt):
        return functools.partial(
            pl.kernel,
            out_type=jax.ShapeDtypeStruct((_BATCH, nt * _DIM), jnp.float32),
            mesh=mesh,
            scratch_types=[
                pltpu.VMEM((nt, _BPW), jnp.int32),
                pltpu.VMEM((_BPW, _DIM), jnp.float32),
                pltpu.VMEM((_BPW, _DIM), jnp.float32),
                pltpu.SemaphoreType.DMA,
                pltpu.SemaphoreType.DMA,
                pltpu.SemaphoreType.DMA,
                pltpu.SemaphoreType.DMA,
            ],
            compiler_params=pltpu.CompilerParams(use_tc_tiling_on_sc=False),
        )(functools.partial(_group_kernel, nt))

    outs = []
    for g in range((_N_CAT + _GROUP - 1) // _GROUP):
        lo = g * _GROUP
        nt = min(_GROUP, _N_CAT - lo)
        outs.append(make_group(nt)(idx[lo:lo + nt], *tabs[lo:lo + nt]))
    return jnp.concatenate(outs, axis=1)


def kernel(x, table_0, table_1, table_2, table_3, table_4, table_5, table_6,
           table_7, table_8, table_9, table_10, table_11, table_12, table_13,
           table_14, table_15, table_16, table_17, table_18, table_19,
           table_20, table_21, table_22, table_23, table_24, table_25):
    tabs = (table_0, table_1, table_2, table_3, table_4, table_5, table_6,
            table_7, table_8, table_9, table_10, table_11, table_12, table_13,
            table_14, table_15, table_16, table_17, table_18, table_19,
            table_20, table_21, table_22, table_23, table_24, table_25)
    idx = x[:, :_N_CAT].astype(jnp.int32).T
    emb = _run(idx, *tabs)
    cont = x[:, _N_CAT:_N_CAT + _N_CONT]
    return emb, cont
